# baseline (speedup 1.0000x reference)
import functools

import jax
import jax.numpy as jnp
from jax import lax
from jax.experimental import pallas as pl
from jax.experimental.pallas import tpu as pltpu
from jax.experimental.pallas import tpu_sc as plsc

R = 128
N = 32768
K = 512
L = 16
NC = 2
NS = 16
NW = NC * NS
ROWS_PER_W = R // NW

SAMPLE_STRIDE = 16
NSAMP = N // L // SAMPLE_STRIDE
KK_SAMP = 64
KEY_NEG_INF = -2139095041

_I32 = jnp.int32
_MAXPOS = 0x7FFFFFFF


def _lanes():
    return lax.iota(_I32, L)


def _scalar(v, i=0):
    return jax.lax.squeeze(jax.lax.slice(v, (i,), (i + 1,)), (0,))


def _popcnt(mask):
    return _scalar(plsc.all_reduce_population_count(mask))


def _bits_to_key(u):
    return jnp.where(u < 0, u ^ _MAXPOS, u)


def _to_key(x_f32):
    return _bits_to_key(plsc.bitcast(x_f32, _I32))


def _from_key(s):
    u = jnp.where(s < 0, s ^ _MAXPOS, s)
    return plsc.bitcast(u, jnp.float32)


def _zero_range(ref, nbins):
    @plsc.parallel_loop(0, nbins // L, step=1, unroll=8)
    def _(i):
        ref[pl.ds(i * L, L)] = jnp.zeros((L,), _I32)


def _find_bin(hist, nbins, kk, start_bin, merged=False):
    lanes = _lanes()
    j0 = (nbins - 1 - start_bin) // L

    def cond(carry):
        j, found, bsel, nab, tot = carry
        return (found == 0) & (j < nbins // L)

    def body(carry):
        j, found, bsel, nab, tot = carry
        base = nbins - (j + 1) * L
        h = hist[pl.ds(base, L)]
        if merged:
            h = h + hist[pl.ds(base + 2048, L)]
        hr = lax.rev(h, (0,))
        c = plsc.cumsum(hr)
        cross = (tot + c) >= kk
        anyc = _popcnt(cross)
        lane = _scalar(plsc.all_reduce_ffs(cross))
        exl = c - hr
        nab_c = tot + jnp.sum(jnp.where(lanes == lane, exl, 0))
        bin_at = base + (L - 1) - lane
        upd = anyc > 0
        bsel = jnp.where(upd, bin_at, bsel)
        nab = jnp.where(upd, nab_c, nab)
        found = jnp.where(upd, 1, 0)
        tot = tot + _scalar(c, L - 1)
        return (j + 1, found, bsel, nab, tot)

    init = (j0, jnp.int32(0), jnp.int32(0), jnp.int32(0), jnp.int32(0))
    _, _, bsel, nab, _ = lax.while_loop(cond, body, init)
    return bsel, nab


def _hist_pass(src, hist, m, shift, mask_bits, convert, bias=0):
    lanes = _lanes()
    nv = (m + (L - 1)) // L

    def body(i, dmax):
        s = src[pl.ds(i * L, L)]
        if convert:
            s = _bits_to_key(s)
        valid = (i * L + lanes) < m
        if bias != 0:
            d = jnp.right_shift(s, shift) + bias
        elif shift != 0:
            d = jnp.right_shift(s, shift) & mask_bits
        else:
            d = s & mask_bits
        cnt, last = plsc.scan_count(d, mask=valid)
        plsc.addupdate_scatter(hist, [d], cnt, mask=last)
        return jnp.maximum(dmax, jnp.where(valid, d, 0))

    dmax = lax.fori_loop(0, nv, body, jnp.zeros((L,), _I32))
    return jnp.max(dmax)


def _hist_pass2(src, hist, m, convert):
    lanes = _lanes()
    nv = (m + (L - 1)) // L
    nv2 = (nv + 1) // 2

    def body(i, dmax):
        for half, ioff in ((0, 0), (1, nv2)):
            idx = i + ioff
            s = src[pl.ds(idx * L, L)]
            if convert:
                s = _bits_to_key(s)
            valid = (idx * L + lanes) < m
            d = jnp.right_shift(s, 21) + 1024
            cnt, last = plsc.scan_count(d + half * 2048, mask=valid)
            plsc.addupdate_scatter(hist, [d + half * 2048], cnt, mask=last)
            dmax = jnp.maximum(dmax, jnp.where(valid, d, 0))
        return dmax

    dmax = lax.fori_loop(0, nv2, body, jnp.zeros((L,), _I32))
    return jnp.max(dmax)


def _split(buf, dbuf, m, d_off, shift, mask_bits, bsel, convert):
    lanes = _lanes()
    nv = (m + (L - 1)) // L

    @plsc.parallel_loop(0, nv, step=1, unroll=4,
                        carry=(d_off, jnp.int32(0)))
    def body(i, carry):
        doff, toff = carry
        s = buf[pl.ds(i * L, L)]
        if convert:
            s = _bits_to_key(s)
        valid = (i * L + lanes) < m
        if mask_bits is None:
            d = jnp.right_shift(s, shift)
        elif shift == 0:
            d = s & mask_bits
        else:
            d = jnp.right_shift(s, shift) & mask_bits
        m_gt = valid & (d > bsel)
        m_eq = valid & (d == bsel)
        plsc.store_compressed(dbuf.at[pl.ds(doff, L)], s, mask=m_gt)
        plsc.store_compressed(buf.at[pl.ds(toff, L)], s, mask=m_eq)
        return (doff + _popcnt(m_gt), toff + _popcnt(m_eq))

    d_off, n_eq = body
    return d_off, n_eq


def _make_kernel():
    mesh = plsc.VectorSubcoreMesh(core_axis_name="c", subcore_axis_name="s",
                                  num_cores=NC, num_subcores=NS)

    @functools.partial(
        pl.kernel,
        out_type=jax.ShapeDtypeStruct((R, K), jnp.float32),
        mesh=mesh,
        scratch_types=[
            pltpu.VMEM((N,), jnp.float32),
            pltpu.VMEM((N + L,), _I32),
            pltpu.VMEM((4096,), _I32),
            pltpu.VMEM((4 * 256,), _I32),
            pltpu.VMEM((4 * (K + L),), _I32),
            pltpu.VMEM((4 * (K + L),), _I32),
            pltpu.VMEM((4 * K,), jnp.float32),
        ],
        compiler_params=pltpu.CompilerParams(needs_layout_passes=False),
    )
    def topk_kernel(in_hbm, out_hbm, row_v, bufF, hist, offs, dD, dD2, out_f):
        wid = lax.axis_index("c") * NS + lax.axis_index("s")
        lanes = _lanes()

        DSTRIDE = K + L

        def do_row(rr, carry):
            r = wid * ROWS_PER_W + rr
            d_base = rr * DSTRIDE
            pltpu.sync_copy(in_hbm.at[r], row_v)

            _zero_range(hist, 4096)

            def samp(i, smax):
                for half, ioff in ((0, 0), (1, NSAMP // 2)):
                    idx = i + ioff
                    x = row_v[pl.ds(idx * (L * SAMPLE_STRIDE)
                                    + (idx & 15) * L, L)]
                    s = _to_key(x)
                    b = jnp.right_shift(s, 21) + 1024 + half * 2048
                    cnt, last = plsc.scan_count(b)
                    plsc.addupdate_scatter(hist, [b], cnt, mask=last)
                    smax = jnp.maximum(smax, s)
                return smax

            smax = lax.fori_loop(0, NSAMP // 2, samp,
                                 jnp.full((L,), -0x80000000, _I32), unroll=4)
            maxbin = jnp.right_shift(jnp.max(smax), 21) + 1024
            bs, _ = _find_bin(hist, 2048, jnp.int32(KK_SAMP), maxbin,
                              merged=True)
            losel = jnp.maximum(lax.shift_left(bs - 1024, 21),
                                jnp.int32(KEY_NEG_INF))
            pivot = _from_key(jnp.full((L,), 1, _I32) * losel)

            @plsc.parallel_loop(0, N // L, step=1, unroll=8,
                                carry=jnp.int32(0))
            def ext(i, off):
                x = row_v[pl.ds(i * L, L)]
                msk = x >= pivot
                plsc.store_compressed(bufF.at[pl.ds(off, L)],
                                      plsc.bitcast(x, _I32), mask=msk)
                return off + _popcnt(msk)

            m1 = ext

            @pl.when(m1 < K)
            def _():
                @plsc.parallel_loop(0, N // L, step=1, unroll=8)
                def _(i):
                    bufF[pl.ds(i * L, L)] = plsc.bitcast(
                        row_v[pl.ds(i * L, L)], _I32)

            m1 = jnp.where(m1 < K, jnp.int32(N), m1)

            _zero_range(hist, 4096)
            dmax1 = _hist_pass2(bufF, hist, m1, True)
            b1, _ = _find_bin(hist, 2048, jnp.int32(K), dmax1, merged=True)
            d_off, mt2 = _split(bufF, dD, m1, d_base, 21, None,
                                b1 - 1024, True)

            k2 = jnp.int32(K) - (d_off - d_base)
            _zero_range(hist, 2048)
            dmax2 = _hist_pass(bufF, hist, mt2, 10, 0x7FF, False)
            b2, _ = _find_bin(hist, 2048, k2, dmax2)
            d_off, mt3 = _split(bufF, dD, mt2, d_off, 10, 0x7FF, b2, False)

            k3 = jnp.int32(K) - (d_off - d_base)
            _zero_range(hist, 1024)
            dmax3 = _hist_pass(bufF, hist, mt3, 0, 0x3FF, False)
            b3, _ = _find_bin(hist, 1024, k3, dmax3)
            d_off, _ = _split(bufF, dD, mt3, d_off, 0, 0x3FF, b3, False)

            k4 = jnp.int32(K) - (d_off - d_base)
            tied = jnp.broadcast_to(_scalar(bufF[pl.ds(0, L)]), (L,))

            def fill(j, off):
                mfill = lanes < (k4 - j * L)
                plsc.store_compressed(dD.at[pl.ds(off, L)], tied, mask=mfill)
                return off + _popcnt(mfill)

            lax.fori_loop(0, (k4 + (L - 1)) // L, fill, d_off)
            return carry

        lax.fori_loop(0, ROWS_PER_W, do_row, 0)

        def radix_pass(src, dst, shift, signed_top):
            _zero_range(hist, 4 * 256)

            def digit(s):
                if signed_top:
                    return jnp.right_shift(s, shift) + 128
                return jnp.right_shift(s, shift) & 0xFF

            def hb(i, c):
                for rr in range(ROWS_PER_W):
                    s = src[pl.ds(rr * DSTRIDE + i * L, L)]
                    d = digit(s) + rr * 256
                    cnt, last = plsc.scan_count(d)
                    plsc.addupdate_scatter(hist, [d], cnt, mask=last)
                return c

            lax.fori_loop(0, K // L, hb, 0, unroll=2)

            def sb(j, tots):
                new = []
                for rr in range(ROWS_PER_W):
                    base = rr * 256 + 256 - (j + 1) * L
                    h = hist[pl.ds(base, L)]
                    hr = lax.rev(h, (0,))
                    exl = tots[rr] + plsc.cumsum(hr) - hr
                    offs[pl.ds(base, L)] = lax.rev(exl, (0,)) + rr * DSTRIDE
                    new.append(tots[rr] + jnp.sum(h))
                return tuple(new)

            lax.fori_loop(0, 256 // L, sb, (jnp.int32(0),) * ROWS_PER_W)

            def pb(i, c):
                for rr in range(ROWS_PER_W):
                    s = src[pl.ds(rr * DSTRIDE + i * L, L)]
                    d = digit(s) + rr * 256
                    cnt, last = plsc.scan_count(d)
                    base = plsc.load_gather(offs, [d])
                    pos = base + cnt - 1
                    plsc.store_scatter(dst, [pos], s)
                    plsc.addupdate_scatter(offs, [d], cnt, mask=last)
                return c

            lax.fori_loop(0, K // L, pb, 0, unroll=2)

        radix_pass(dD, dD2, 0, False)
        radix_pass(dD2, dD, 8, False)
        radix_pass(dD, dD2, 16, False)
        radix_pass(dD2, dD, 24, True)

        @plsc.parallel_loop(0, ROWS_PER_W * (K // L), step=1, unroll=8)
        def _(i):
            src_off = (i + i // (K // L)) * L
            out_f[pl.ds(i * L, L)] = _from_key(dD[pl.ds(src_off, L)])

        for rr in range(ROWS_PER_W):
            pltpu.sync_copy(out_f.at[pl.ds(rr * K, K)],
                            out_hbm.at[wid * ROWS_PER_W + rr])

    return topk_kernel


def kernel(input):
    return _make_kernel()(input)


# speedup vs baseline: 1.0054x; 1.0054x over previous
import functools

import jax
import jax.numpy as jnp
from jax import lax
from jax.experimental import pallas as pl
from jax.experimental.pallas import tpu as pltpu
from jax.experimental.pallas import tpu_sc as plsc

R = 128
N = 32768
K = 512
L = 16
NC = 2
NS = 16
NW = NC * NS
ROWS_PER_W = R // NW

SAMPLE_STRIDE = 16
NSAMP = N // L // SAMPLE_STRIDE
KK_SAMP = 64
KEY_NEG_INF = -2139095041

_I32 = jnp.int32
_MAXPOS = 0x7FFFFFFF


def _lanes():
    return lax.iota(_I32, L)


def _scalar(v, i=0):
    return jax.lax.squeeze(jax.lax.slice(v, (i,), (i + 1,)), (0,))


def _popcnt(mask):
    return _scalar(plsc.all_reduce_population_count(mask))


def _bits_to_key(u):
    return jnp.where(u < 0, u ^ _MAXPOS, u)


def _to_key(x_f32):
    return _bits_to_key(plsc.bitcast(x_f32, _I32))


def _from_key(s):
    u = jnp.where(s < 0, s ^ _MAXPOS, s)
    return plsc.bitcast(u, jnp.float32)


def _zero_range(ref, nbins):
    @plsc.parallel_loop(0, nbins // L, step=1, unroll=8)
    def _(i):
        ref[pl.ds(i * L, L)] = jnp.zeros((L,), _I32)


def _find_bin(hist, nbins, kk, start_bin):
    lanes = _lanes()
    j0 = (nbins - 1 - start_bin) // L

    def cond(carry):
        j, found, bsel, nab, tot = carry
        return (found == 0) & (j < nbins // L)

    def body(carry):
        j, found, bsel, nab, tot = carry
        base = nbins - (j + 1) * L
        h = hist[pl.ds(base, L)]
        hr = lax.rev(h, (0,))
        c = plsc.cumsum(hr)
        cross = (tot + c) >= kk
        anyc = _popcnt(cross)
        lane = _scalar(plsc.all_reduce_ffs(cross))
        exl = c - hr
        nab_c = tot + jnp.sum(jnp.where(lanes == lane, exl, 0))
        bin_at = base + (L - 1) - lane
        upd = anyc > 0
        bsel = jnp.where(upd, bin_at, bsel)
        nab = jnp.where(upd, nab_c, nab)
        found = jnp.where(upd, 1, 0)
        tot = tot + _scalar(c, L - 1)
        return (j + 1, found, bsel, nab, tot)

    init = (j0, jnp.int32(0), jnp.int32(0), jnp.int32(0), jnp.int32(0))
    _, _, bsel, nab, _ = lax.while_loop(cond, body, init)
    return bsel, nab


def _hist_pass(src, hist, m, shift, mask_bits, convert, bias=0):
    lanes = _lanes()
    nv = (m + (L - 1)) // L

    def body(i, dmax):
        s = src[pl.ds(i * L, L)]
        if convert:
            s = _bits_to_key(s)
        valid = (i * L + lanes) < m
        if bias != 0:
            d = jnp.right_shift(s, shift) + bias
        elif shift != 0:
            d = jnp.right_shift(s, shift) & mask_bits
        else:
            d = s & mask_bits
        cnt, last = plsc.scan_count(d, mask=valid)
        plsc.addupdate_scatter(hist, [d], cnt, mask=last)
        return jnp.maximum(dmax, jnp.where(valid, d, 0))

    dmax = lax.fori_loop(0, nv, body, jnp.zeros((L,), _I32))
    return jnp.max(dmax)


def _split(buf, dbuf, m, d_off, shift, mask_bits, bsel, convert):
    lanes = _lanes()
    nv = (m + (L - 1)) // L

    @plsc.parallel_loop(0, nv, step=1, unroll=4,
                        carry=(d_off, jnp.int32(0)))
    def body(i, carry):
        doff, toff = carry
        s = buf[pl.ds(i * L, L)]
        if convert:
            s = _bits_to_key(s)
        valid = (i * L + lanes) < m
        if mask_bits is None:
            d = jnp.right_shift(s, shift)
        elif shift == 0:
            d = s & mask_bits
        else:
            d = jnp.right_shift(s, shift) & mask_bits
        m_gt = valid & (d > bsel)
        m_eq = valid & (d == bsel)
        plsc.store_compressed(dbuf.at[pl.ds(doff, L)], s, mask=m_gt)
        plsc.store_compressed(buf.at[pl.ds(toff, L)], s, mask=m_eq)
        return (doff + _popcnt(m_gt), toff + _popcnt(m_eq))

    d_off, n_eq = body
    return d_off, n_eq


def _make_kernel():
    mesh = plsc.VectorSubcoreMesh(core_axis_name="c", subcore_axis_name="s",
                                  num_cores=NC, num_subcores=NS)

    @functools.partial(
        pl.kernel,
        out_type=jax.ShapeDtypeStruct((R, K), jnp.float32),
        mesh=mesh,
        scratch_types=[
            pltpu.VMEM((N,), jnp.float32),
            pltpu.VMEM((N + L,), _I32),
            pltpu.VMEM((2048,), _I32),
            pltpu.VMEM((4 * 256,), _I32),
            pltpu.VMEM((4 * (K + L),), _I32),
            pltpu.VMEM((4 * (K + L),), _I32),
            pltpu.VMEM((4 * K,), jnp.float32),
        ],
        compiler_params=pltpu.CompilerParams(needs_layout_passes=False),
    )
    def topk_kernel(in_hbm, out_hbm, row_v, bufF, hist, offs, dD, dD2, out_f):
        wid = lax.axis_index("c") * NS + lax.axis_index("s")
        lanes = _lanes()

        DSTRIDE = K + L

        def do_row(rr, carry):
            r = wid * ROWS_PER_W + rr
            d_base = rr * DSTRIDE
            pltpu.sync_copy(in_hbm.at[r], row_v)

            _zero_range(hist, 2048)

            def samp(i, smax):
                x = row_v[pl.ds(i * (L * SAMPLE_STRIDE) + (i & 15) * L, L)]
                s = _to_key(x)
                b = jnp.right_shift(s, 21) + 1024
                cnt, last = plsc.scan_count(b)
                plsc.addupdate_scatter(hist, [b], cnt, mask=last)
                return jnp.maximum(smax, s)

            smax = lax.fori_loop(0, NSAMP, samp,
                                 jnp.full((L,), -0x80000000, _I32), unroll=8)
            maxbin = jnp.right_shift(jnp.max(smax), 21) + 1024
            bs, _ = _find_bin(hist, 2048, jnp.int32(KK_SAMP), maxbin)
            losel = jnp.maximum(lax.shift_left(bs - 1024, 21),
                                jnp.int32(KEY_NEG_INF))
            pivot = _from_key(jnp.full((L,), 1, _I32) * losel)

            @plsc.parallel_loop(0, N // L, step=1, unroll=8,
                                carry=jnp.int32(0))
            def ext(i, off):
                x = row_v[pl.ds(i * L, L)]
                msk = x >= pivot
                plsc.store_compressed(bufF.at[pl.ds(off, L)],
                                      plsc.bitcast(x, _I32), mask=msk)
                return off + _popcnt(msk)

            m1 = ext

            @pl.when(m1 < K)
            def _():
                @plsc.parallel_loop(0, N // L, step=1, unroll=8)
                def _(i):
                    bufF[pl.ds(i * L, L)] = plsc.bitcast(
                        row_v[pl.ds(i * L, L)], _I32)

            m1 = jnp.where(m1 < K, jnp.int32(N), m1)

            _zero_range(hist, 2048)
            dmax1 = _hist_pass(bufF, hist, m1, 21, None, True, bias=1024)
            b1, _ = _find_bin(hist, 2048, jnp.int32(K), dmax1)
            d_off, mt2 = _split(bufF, dD, m1, d_base, 21, None,
                                b1 - 1024, True)

            k2 = jnp.int32(K) - (d_off - d_base)
            _zero_range(hist, 2048)
            dmax2 = _hist_pass(bufF, hist, mt2, 10, 0x7FF, False)
            b2, _ = _find_bin(hist, 2048, k2, dmax2)
            d_off, mt3 = _split(bufF, dD, mt2, d_off, 10, 0x7FF, b2, False)

            k3 = jnp.int32(K) - (d_off - d_base)
            _zero_range(hist, 1024)
            dmax3 = _hist_pass(bufF, hist, mt3, 0, 0x3FF, False)
            b3, _ = _find_bin(hist, 1024, k3, dmax3)
            d_off, _ = _split(bufF, dD, mt3, d_off, 0, 0x3FF, b3, False)

            k4 = jnp.int32(K) - (d_off - d_base)
            tied = jnp.broadcast_to(_scalar(bufF[pl.ds(0, L)]), (L,))

            def fill(j, off):
                mfill = lanes < (k4 - j * L)
                plsc.store_compressed(dD.at[pl.ds(off, L)], tied, mask=mfill)
                return off + _popcnt(mfill)

            lax.fori_loop(0, (k4 + (L - 1)) // L, fill, d_off)
            return carry

        lax.fori_loop(0, ROWS_PER_W, do_row, 0)

        def radix_pass(src, dst, shift, signed_top):
            _zero_range(hist, 4 * 256)

            def digit(s):
                if signed_top:
                    return jnp.right_shift(s, shift) + 128
                return jnp.right_shift(s, shift) & 0xFF

            def hb(i, c):
                for rr in range(ROWS_PER_W):
                    s = src[pl.ds(rr * DSTRIDE + i * L, L)]
                    d = digit(s) + rr * 256
                    cnt, last = plsc.scan_count(d)
                    plsc.addupdate_scatter(hist, [d], cnt, mask=last)
                return c

            lax.fori_loop(0, K // L, hb, 0, unroll=2)

            def sb(j, tots):
                new = []
                for rr in range(ROWS_PER_W):
                    base = rr * 256 + 256 - (j + 1) * L
                    h = hist[pl.ds(base, L)]
                    hr = lax.rev(h, (0,))
                    exl = tots[rr] + plsc.cumsum(hr) - hr
                    offs[pl.ds(base, L)] = lax.rev(exl, (0,)) + rr * DSTRIDE
                    new.append(tots[rr] + jnp.sum(h))
                return tuple(new)

            lax.fori_loop(0, 256 // L, sb, (jnp.int32(0),) * ROWS_PER_W)

            def pb(i, c):
                for rr in range(ROWS_PER_W):
                    s = src[pl.ds(rr * DSTRIDE + i * L, L)]
                    d = digit(s) + rr * 256
                    cnt, last = plsc.scan_count(d)
                    base = plsc.load_gather(offs, [d])
                    pos = base + cnt - 1
                    plsc.store_scatter(dst, [pos], s)
                    plsc.addupdate_scatter(offs, [d], cnt, mask=last)
                return c

            lax.fori_loop(0, K // L, pb, 0, unroll=2)

        radix_pass(dD, dD2, 0, False)
        radix_pass(dD2, dD, 8, False)
        radix_pass(dD, dD2, 16, False)
        radix_pass(dD2, dD, 24, True)

        @plsc.parallel_loop(0, ROWS_PER_W * (K // L), step=1, unroll=8)
        def _(i):
            src_off = (i + i // (K // L)) * L
            out_f[pl.ds(i * L, L)] = _from_key(dD[pl.ds(src_off, L)])

        for rr in range(ROWS_PER_W):
            pltpu.sync_copy(out_f.at[pl.ds(rr * K, K)],
                            out_hbm.at[wid * ROWS_PER_W + rr])

    return topk_kernel


def kernel(input):
    return _make_kernel()(input)
